# baseline (speedup 1.0000x reference)
import functools

import jax
import jax.numpy as jnp
from jax import lax
from jax.experimental import pallas as pl
from jax.experimental.pallas import tpu as pltpu
from jax.experimental.pallas import tpu_sc as plsc

_NC = 2
_NS = 16
_NW = _NC * _NS
_L = 16


def _sc_index_build(adjx, nodes32, *, B, H, N):
    BPW = B // _NW
    GPW = BPW * H
    FPW = GPW * H

    mesh = plsc.VectorSubcoreMesh(core_axis_name="c", subcore_axis_name="s")

    @functools.partial(
        pl.kernel,
        out_type=jax.ShapeDtypeStruct((_NW * FPW,), jnp.int32),
        mesh=mesh,
        compiler_params=pltpu.CompilerParams(
            needs_layout_passes=False, use_tc_tiling_on_sc=False),
        scratch_types=[
            pltpu.VMEM((BPW,), jnp.int32),
            pltpu.VMEM((BPW, 16), jnp.int32),
            pltpu.VMEM((BPW * 16,), jnp.int32),
            pltpu.VMEM((BPW * 16, 16), jnp.int32),
            pltpu.VMEM((FPW,), jnp.int32),
            pltpu.SemaphoreType.DMA,
        ],
    )
    def ka(adjx_hbm, nodes_hbm, fidx_hbm,
           nodes_v, nb2x_v, hood2p_v, nb1x_v, fidx_v, sem):
        wid = lax.axis_index("s") * _NC + lax.axis_index("c")

        pltpu.sync_copy(nodes_hbm.at[pl.ds(wid * BPW, BPW)], nodes_v)

        pltpu.async_copy(adjx_hbm.at[nodes_v], nb2x_v, sem).wait()

        def h2_body(kk, _):
            hood2p_v[pl.ds(kk * 16, 16)] = nb2x_v[kk, :]
            return 0

        lax.fori_loop(0, BPW, h2_body, 0)

        pltpu.async_copy(adjx_hbm.at[hood2p_v], nb1x_v, sem).wait()

        iot = lax.iota(jnp.int32, 16)
        msk = iot < H

        def fx_body(kk, _):
            for i in range(H):
                v = nb1x_v[kk * 16 + i, :]
                pos = iot * GPW + (kk * H + i)
                plsc.store_scatter(fidx_v, [pos], v, mask=msk)
            return 0

        lax.fori_loop(0, BPW, fx_body, 0)

        pltpu.sync_copy(fidx_v, fidx_hbm.at[pl.ds(wid * FPW, FPW)])

    return ka(adjx, nodes32)


def _sc_segment_sum(features, fidx, *, B, H, D):
    G = B * H
    GPW = G // _NW
    FPW = GPW * H

    mesh = plsc.VectorSubcoreMesh(core_axis_name="c", subcore_axis_name="s")

    @functools.partial(
        pl.kernel,
        out_type=jax.ShapeDtypeStruct((G, D), jnp.float32),
        mesh=mesh,
        compiler_params=pltpu.CompilerParams(needs_layout_passes=False),
        scratch_types=[
            pltpu.VMEM((FPW,), jnp.int32),
            pltpu.VMEM((GPW, D), jnp.float32),
            pltpu.SemaphoreType.DMA,
        ],
    )
    def kb(features_hbm, fidx_hbm, out_hbm, fidx_v, acc_v, sem):
        wid = lax.axis_index("s") * _NC + lax.axis_index("c")

        pltpu.sync_copy(fidx_hbm.at[pl.ds(wid * FPW, FPW)], fidx_v)

        pltpu.async_copy(
            features_hbm.at[fidx_v.at[pl.ds(0, GPW)]], acc_v, sem).wait()
        for j in range(1, H):
            pltpu.async_copy(
                features_hbm.at[fidx_v.at[pl.ds(j * GPW, GPW)]], acc_v, sem,
                add=True).wait()

        pltpu.sync_copy(acc_v, out_hbm.at[pl.ds(wid * GPW, GPW)])

    return kb(features, fidx)


def _tc_encode(agg1, W1t, W2t, *, B, H, D, E):
    BBLK = 512
    RBLK = BBLK * H

    def body(x_ref, w1t_ref, w2t_ref, out_ref):
        x = x_ref[...]
        h1 = jnp.maximum(
            jnp.dot(x, w1t_ref[...], preferred_element_type=jnp.float32), 0.0)
        a2 = jnp.sum(h1.reshape(BBLK, H, E), axis=1)
        h2 = jnp.maximum(
            jnp.dot(a2, w2t_ref[...], preferred_element_type=jnp.float32), 0.0)
        out_ref[...] = h2.T

    return pl.pallas_call(
        body,
        grid=(B // BBLK,),
        in_specs=[
            pl.BlockSpec((RBLK, D), lambda i: (i, 0)),
            pl.BlockSpec((D, E), lambda i: (0, 0)),
            pl.BlockSpec((E, E), lambda i: (0, 0)),
        ],
        out_specs=pl.BlockSpec((E, BBLK), lambda i: (0, i)),
        out_shape=jax.ShapeDtypeStruct((E, B), jnp.float32),
    )(agg1, W1t, W2t)


def kernel(features, adj, nodes, W1, W2):
    N, S = adj.shape
    B = nodes.shape[0]
    D = features.shape[1]
    E = W1.shape[0]
    H = S + 1

    adj32 = adj.astype(jnp.int32)
    selfcol = jnp.arange(N, dtype=jnp.int32)[:, None]
    adjx = jnp.concatenate(
        [adj32, selfcol, jnp.zeros((N, 16 - S - 1), jnp.int32)], axis=1)
    nodes32 = nodes.astype(jnp.int32)

    fidx = _sc_index_build(adjx, nodes32, B=B, H=H, N=N)
    agg1 = _sc_segment_sum(features, fidx, B=B, H=H, D=D)
    return _tc_encode(agg1, W1.T / H, W2.T / H, B=B, H=H, D=D, E=E)
